# Initial kernel scaffold; baseline (speedup 1.0000x reference)
import jax
import jax.numpy as jnp
from jax.experimental import pallas as pl


def kernel(query, key, value, key_cache, value_cache, slot_mapping, block_tables, context_lens):
    raise NotImplementedError("write your pallas kernel here")


# speedup vs baseline: 1.8605x; 1.8605x over previous
import functools

import jax
import jax.numpy as jnp
from jax import lax
from jax.experimental import pallas as pl
from jax.experimental.pallas import tpu as pltpu

_SCALE = 0.08838834764831845
_H = 16
_D = 128
_NP = 2
_PL = 2048
_NG = 16
_BS = 16
_X = 8
_NB = 512
_MC = 1024
_HD = _H * _D

_BQ = 512
_NQ = _PL // _BQ


def _prompt_body(q_ref, k_ref, v_ref, o_ref, acc_ref, m_ref, l_ref):
    qb = pl.program_id(1)
    kb = pl.program_id(2)

    @pl.when(kb == 0)
    def _init():
        m_ref[...] = jnp.full_like(m_ref, -1e30)
        l_ref[...] = jnp.zeros_like(l_ref)
        acc_ref[...] = jnp.zeros_like(acc_ref)

    @pl.when(kb <= qb)
    def _compute():
        row = qb * _BQ + lax.broadcasted_iota(jnp.int32, (_BQ, _BQ), 0)
        col = kb * _BQ + lax.broadcasted_iota(jnp.int32, (_BQ, _BQ), 1)
        neg = jnp.where(col > row, -100000.0, 0.0)
        for h in range(_H):
            sl = slice(h * _D, (h + 1) * _D)
            qh = q_ref[0, :, sl] * _SCALE
            kh = k_ref[0, :, sl]
            vh = v_ref[0, :, sl]
            s = lax.dot_general(qh, kh, (((1,), (1,)), ((), ())),
                                preferred_element_type=jnp.float32)
            s = s + neg
            m_prev = m_ref[:, h:h + 1]
            l_prev = l_ref[:, h:h + 1]
            m_new = jnp.maximum(m_prev, jnp.max(s, axis=1, keepdims=True))
            p = jnp.exp(s - m_new)
            alpha = jnp.exp(m_prev - m_new)
            l_new = alpha * l_prev + jnp.sum(p, axis=1, keepdims=True)
            m_ref[:, h:h + 1] = m_new
            l_ref[:, h:h + 1] = l_new
            pv = lax.dot_general(p, vh, (((1,), (0,)), ((), ())),
                                 preferred_element_type=jnp.float32)
            acc_ref[:, sl] = acc_ref[:, sl] * alpha + pv

    @pl.when(kb == qb)
    def _finalize():
        for h in range(_H):
            sl = slice(h * _D, (h + 1) * _D)
            o_ref[0, :, sl] = acc_ref[:, sl] / l_ref[:, h:h + 1]


def _prompt_attention(qp, kp, vp):
    grid = (_NP, _NQ, _NQ)
    qspec = pl.BlockSpec((1, _BQ, _HD), lambda p, qb, kb: (p, qb, 0))
    kspec = pl.BlockSpec((1, _BQ, _HD),
                         lambda p, qb, kb: (p, jnp.minimum(kb, qb), 0))
    return pl.pallas_call(
        _prompt_body,
        grid=grid,
        in_specs=[qspec, kspec, kspec],
        out_specs=qspec,
        out_shape=jax.ShapeDtypeStruct((_NP, _PL, _HD), jnp.float32),
        scratch_shapes=[
            pltpu.VMEM((_BQ, _HD), jnp.float32),
            pltpu.VMEM((_BQ, _H), jnp.float32),
            pltpu.VMEM((_BQ, _H), jnp.float32),
        ],
    )(qp, kp, vp)


def _cache_body(inv_ref, valid_ref, kc_ref, vc_ref, *rest):
    krow_refs = rest[:_BS]
    vrow_refs = rest[_BS:2 * _BS]
    ko_ref, vo_ref = rest[2 * _BS], rest[2 * _BS + 1]
    b = pl.program_id(0)
    ko_ref[0] = kc_ref[0]
    vo_ref[0] = vc_ref[0]
    for j in range(_BS):
        ok = valid_ref[b * _BS + j] > 0

        @pl.when(ok)
        def _write(j=j):
            ko_ref[0, :, :, j, :] = krow_refs[j][0, 0].reshape(_H, _D // _X, _X)
            vo_ref[0, :, j, :] = vrow_refs[j][0, 0].reshape(_H, _D)


def _cache_update(key_cache, value_cache, k_new, v_new, inv, valid):
    kc_spec = pl.BlockSpec((1, _H, _D // _X, _BS, _X),
                           lambda b, inv_r, val_r: (b, 0, 0, 0, 0))
    vc_spec = pl.BlockSpec((1, _H, _BS, _D),
                           lambda b, inv_r, val_r: (b, 0, 0, 0))

    def _row_idx(b, inv_r, val_r, jj):
        return (inv_r[b * _BS + jj], 0, 0)

    row_specs = [pl.BlockSpec((1, 1, _HD), functools.partial(_row_idx, jj=j))
                 for j in range(_BS)]
    k_new = k_new.reshape(-1, 1, _HD)
    v_new = v_new.reshape(-1, 1, _HD)
    grid_spec = pltpu.PrefetchScalarGridSpec(
        num_scalar_prefetch=2,
        grid=(_NB,),
        in_specs=[kc_spec, vc_spec] + row_specs + row_specs,
        out_specs=[kc_spec, vc_spec],
    )
    return pl.pallas_call(
        _cache_body,
        grid_spec=grid_spec,
        out_shape=[
            jax.ShapeDtypeStruct(key_cache.shape, jnp.float32),
            jax.ShapeDtypeStruct(value_cache.shape, jnp.float32),
        ],
    )(inv, valid, key_cache, value_cache,
      *([k_new] * _BS), *([v_new] * _BS))


_NT = _MC // _BS


def _gen_body(bt_ref, cl_ref, q_ref, kc_ref, vc_ref, o_ref,
              acc_ref, m_ref, l_ref):
    g = pl.program_id(0)
    t = pl.program_id(1)

    @pl.when(t == 0)
    def _init():
        m_ref[...] = jnp.full_like(m_ref, -1e30)
        l_ref[...] = jnp.zeros_like(l_ref)
        acc_ref[...] = jnp.zeros_like(acc_ref)

    cl = cl_ref[g]
    tlast = (cl - 1) // _BS

    @pl.when(t <= tlast)
    def _compute():
        q = q_ref[0] * _SCALE
        kblk = kc_ref[0]
        vblk = vc_ref[0]
        s = jnp.zeros((_H, _BS), jnp.float32)
        for c in range(_D // _X):
            qc = q[:, c * _X:(c + 1) * _X]
            kc = kblk[:, c, :, :]
            s = s + jnp.sum(qc[:, None, :] * kc, axis=2)
        pos = t * _BS + lax.broadcasted_iota(jnp.int32, (_H, _BS), 1)
        s = s + jnp.where(pos < cl, 0.0, -100000.0)
        m_prev = m_ref[:, :1]
        l_prev = l_ref[:, :1]
        m_new = jnp.maximum(m_prev, jnp.max(s, axis=1, keepdims=True))
        p = jnp.exp(s - m_new)
        alpha = jnp.exp(m_prev - m_new)
        l_new = alpha * l_prev + jnp.sum(p, axis=1, keepdims=True)
        m_ref[...] = jnp.broadcast_to(m_new, m_ref.shape)
        l_ref[...] = jnp.broadcast_to(l_new, l_ref.shape)
        pv = jnp.sum(p[:, :, None] * vblk, axis=1)
        acc_ref[...] = acc_ref[...] * alpha + pv

    @pl.when(t == _NT - 1)
    def _finalize():
        o_ref[0] = acc_ref[...] / l_ref[:, :1]


def _gen_attention(qg, key_cache, value_cache, block_tables, context_lens):
    def kidx(g, t, bt_r, cl_r):
        te = jnp.minimum(t, (cl_r[g] - 1) // _BS)
        return (bt_r[g, te], 0, 0, 0, 0)

    def vidx(g, t, bt_r, cl_r):
        te = jnp.minimum(t, (cl_r[g] - 1) // _BS)
        return (bt_r[g, te], 0, 0, 0)

    grid_spec = pltpu.PrefetchScalarGridSpec(
        num_scalar_prefetch=2,
        grid=(_NG, _NT),
        in_specs=[
            pl.BlockSpec((1, _H, _D), lambda g, t, bt_r, cl_r: (g, 0, 0)),
            pl.BlockSpec((1, _H, _D // _X, _BS, _X), kidx),
            pl.BlockSpec((1, _H, _BS, _D), vidx),
        ],
        out_specs=pl.BlockSpec((1, _H, _D),
                               lambda g, t, bt_r, cl_r: (g, 0, 0)),
        scratch_shapes=[
            pltpu.VMEM((_H, _D), jnp.float32),
            pltpu.VMEM((_H, 128), jnp.float32),
            pltpu.VMEM((_H, 128), jnp.float32),
        ],
    )
    return pl.pallas_call(
        _gen_body,
        grid_spec=grid_spec,
        out_shape=jax.ShapeDtypeStruct((_NG, _H, _D), jnp.float32),
    )(block_tables, context_lens, qg, key_cache, value_cache)


def kernel(query, key, value, key_cache, value_cache, slot_mapping,
           block_tables, context_lens):
    n_tok = query.shape[0]
    start = _NP * _PL

    qp = query[:start].reshape(_NP, _PL, _HD)
    kp = key[:start].reshape(_NP, _PL, _HD)
    vp = value[:start].reshape(_NP, _PL, _HD)
    out_p = _prompt_attention(qp, kp, vp).reshape(start, _HD)

    nslots = _NB * _BS
    inv = jnp.zeros((nslots,), jnp.int32).at[slot_mapping].set(
        jnp.arange(n_tok, dtype=jnp.int32))
    valid = jnp.zeros((nslots,), jnp.int32).at[slot_mapping].set(1)

    kc_upd, vc_upd = _cache_update(key_cache, value_cache, key, value,
                                   inv, valid)

    qg = query[start:].reshape(_NG, _H, _D)
    out_g = _gen_attention(qg, kc_upd, vc_upd,
                           block_tables.astype(jnp.int32), context_lens)

    return jnp.concatenate([out_p, out_g.reshape(_NG, _HD)], axis=0)


# speedup vs baseline: 3.3209x; 1.7850x over previous
import functools

import jax
import jax.numpy as jnp
from jax import lax
from jax.experimental import pallas as pl
from jax.experimental.pallas import tpu as pltpu

_SCALE = 0.08838834764831845
_H = 16
_D = 128
_NP = 2
_PL = 2048
_NG = 16
_BS = 16
_X = 8
_NB = 512
_MC = 1024
_HD = _H * _D

_BQ = 512
_NQ = _PL // _BQ


def _prompt_body(q_ref, k_ref, v_ref, o_ref, acc_ref, m_ref, l_ref):
    qb = pl.program_id(1)
    kb = pl.program_id(2)

    @pl.when(kb == 0)
    def _init():
        m_ref[...] = jnp.full_like(m_ref, -1e30)
        l_ref[...] = jnp.zeros_like(l_ref)
        acc_ref[...] = jnp.zeros_like(acc_ref)

    @pl.when(kb <= qb)
    def _compute():
        row = qb * _BQ + lax.broadcasted_iota(jnp.int32, (_BQ, _BQ), 0)
        col = kb * _BQ + lax.broadcasted_iota(jnp.int32, (_BQ, _BQ), 1)
        neg = jnp.where(col > row, -100000.0, 0.0)
        for h in range(_H):
            sl = slice(h * _D, (h + 1) * _D)
            qh = q_ref[0, :, sl] * _SCALE
            kh = k_ref[0, :, sl]
            vh = v_ref[0, :, sl]
            s = lax.dot_general(qh, kh, (((1,), (1,)), ((), ())),
                                preferred_element_type=jnp.float32)
            s = s + neg
            m_prev = m_ref[:, h:h + 1]
            l_prev = l_ref[:, h:h + 1]
            m_new = jnp.maximum(m_prev, jnp.max(s, axis=1, keepdims=True))
            p = jnp.exp(s - m_new)
            alpha = jnp.exp(m_prev - m_new)
            l_new = alpha * l_prev + jnp.sum(p, axis=1, keepdims=True)
            m_ref[:, h:h + 1] = m_new
            l_ref[:, h:h + 1] = l_new
            pv = lax.dot_general(p, vh, (((1,), (0,)), ((), ())),
                                 preferred_element_type=jnp.float32)
            acc_ref[:, sl] = acc_ref[:, sl] * alpha + pv

    @pl.when(kb == qb)
    def _finalize():
        for h in range(_H):
            sl = slice(h * _D, (h + 1) * _D)
            o_ref[0, :, sl] = acc_ref[:, sl] / l_ref[:, h:h + 1]


def _prompt_attention(qp, kp, vp):
    grid = (_NP, _NQ, _NQ)
    qspec = pl.BlockSpec((1, _BQ, _HD), lambda p, qb, kb: (p, qb, 0))
    kspec = pl.BlockSpec((1, _BQ, _HD),
                         lambda p, qb, kb: (p, jnp.minimum(kb, qb), 0))
    return pl.pallas_call(
        _prompt_body,
        grid=grid,
        in_specs=[qspec, kspec, kspec],
        out_specs=qspec,
        out_shape=jax.ShapeDtypeStruct((_NP, _PL, _HD), jnp.float32),
        scratch_shapes=[
            pltpu.VMEM((_BQ, _HD), jnp.float32),
            pltpu.VMEM((_BQ, _H), jnp.float32),
            pltpu.VMEM((_BQ, _H), jnp.float32),
        ],
    )(qp, kp, vp)


def _cache_body(inv_ref, valid_ref, kc_ref, vc_ref, *rest):
    krow_refs = rest[:_BS]
    vrow_refs = rest[_BS:2 * _BS]
    ko_ref, vo_ref = rest[2 * _BS], rest[2 * _BS + 1]
    b = pl.program_id(0)
    ko_ref[0] = kc_ref[0].transpose(0, 2, 1, 3).reshape(_H, _BS, _D)
    vo_ref[0] = vc_ref[0]
    for j in range(_BS):
        ok = valid_ref[b * _BS + j] > 0

        @pl.when(ok)
        def _write(j=j):
            ko_ref[0, :, j, :] = krow_refs[j][0, 0].reshape(_H, _D)
            vo_ref[0, :, j, :] = vrow_refs[j][0, 0].reshape(_H, _D)


def _cache_update(key_cache, value_cache, k_new, v_new, inv, valid):
    kc_spec = pl.BlockSpec((1, _H, _D // _X, _BS, _X),
                           lambda b, inv_r, val_r: (b, 0, 0, 0, 0))
    vc_spec = pl.BlockSpec((1, _H, _BS, _D),
                           lambda b, inv_r, val_r: (b, 0, 0, 0))
    ko_spec = pl.BlockSpec((1, _H, _BS, _D),
                           lambda b, inv_r, val_r: (b, 0, 0, 0))

    def _row_idx(b, inv_r, val_r, jj):
        return (inv_r[b * _BS + jj], 0, 0)

    row_specs = [pl.BlockSpec((1, 1, _HD), functools.partial(_row_idx, jj=j))
                 for j in range(_BS)]
    k_new = k_new.reshape(-1, 1, _HD)
    v_new = v_new.reshape(-1, 1, _HD)
    grid_spec = pltpu.PrefetchScalarGridSpec(
        num_scalar_prefetch=2,
        grid=(_NB,),
        in_specs=[kc_spec, vc_spec] + row_specs + row_specs,
        out_specs=[ko_spec, ko_spec],
    )
    return pl.pallas_call(
        _cache_body,
        grid_spec=grid_spec,
        out_shape=[
            jax.ShapeDtypeStruct((_NB, _H, _BS, _D), jnp.float32),
            jax.ShapeDtypeStruct(value_cache.shape, jnp.float32),
        ],
    )(inv, valid, key_cache, value_cache,
      *([k_new] * _BS), *([v_new] * _BS))


_TB = 4
_NT = _MC // (_BS * _TB)


def _gen_body(bt_ref, cl_ref, q_ref, *rest):
    k_refs = rest[:_TB]
    v_refs = rest[_TB:2 * _TB]
    o_ref = rest[2 * _TB]
    acc_ref, m_ref, l_ref = rest[2 * _TB + 1:]
    g = pl.program_id(0)
    tb = pl.program_id(1)

    @pl.when(tb == 0)
    def _init():
        m_ref[...] = jnp.full_like(m_ref, -1e30)
        l_ref[...] = jnp.zeros_like(l_ref)
        acc_ref[...] = jnp.zeros_like(acc_ref)

    cl = cl_ref[g]
    tlast = (cl - 1) // _BS

    @pl.when(tb * _TB <= tlast)
    def _compute():
        q = q_ref[0] * _SCALE
        w = _TB * _BS
        parts = [jnp.sum(q[:, None, :] * k_refs[u][0], axis=2)
                 for u in range(_TB)]
        s = jnp.concatenate(parts, axis=1)
        pos = tb * w + lax.broadcasted_iota(jnp.int32, (_H, w), 1)
        s = s + jnp.where(pos < cl, 0.0, -100000.0)
        m_prev = m_ref[:, :1]
        l_prev = l_ref[:, :1]
        m_new = jnp.maximum(m_prev, jnp.max(s, axis=1, keepdims=True))
        p = jnp.exp(s - m_new)
        alpha = jnp.exp(m_prev - m_new)
        l_new = alpha * l_prev + jnp.sum(p, axis=1, keepdims=True)
        m_ref[...] = jnp.broadcast_to(m_new, m_ref.shape)
        l_ref[...] = jnp.broadcast_to(l_new, l_ref.shape)
        pv = jnp.zeros((_H, _D), jnp.float32)
        for u in range(_TB):
            pv = pv + jnp.sum(p[:, u * _BS:(u + 1) * _BS, None]
                              * v_refs[u][0], axis=1)
        acc_ref[...] = acc_ref[...] * alpha + pv

    @pl.when(tb == _NT - 1)
    def _finalize():
        o_ref[0] = acc_ref[...] / l_ref[:, :1]


def _gen_attention(qg, key_cache, value_cache, block_tables, context_lens):
    def _blk_idx(g, tb, bt_r, cl_r, uu):
        te = jnp.minimum(tb * _TB + uu, (cl_r[g] - 1) // _BS)
        return (bt_r[g, te], 0, 0, 0)

    blk_specs = [pl.BlockSpec((1, _H, _BS, _D),
                              functools.partial(_blk_idx, uu=u))
                 for u in range(_TB)]
    grid_spec = pltpu.PrefetchScalarGridSpec(
        num_scalar_prefetch=2,
        grid=(_NG, _NT),
        in_specs=[pl.BlockSpec((1, _H, _D),
                               lambda g, tb, bt_r, cl_r: (g, 0, 0))]
                 + blk_specs + blk_specs,
        out_specs=pl.BlockSpec((1, _H, _D),
                               lambda g, tb, bt_r, cl_r: (g, 0, 0)),
        scratch_shapes=[
            pltpu.VMEM((_H, _D), jnp.float32),
            pltpu.VMEM((_H, 128), jnp.float32),
            pltpu.VMEM((_H, 128), jnp.float32),
        ],
    )
    return pl.pallas_call(
        _gen_body,
        grid_spec=grid_spec,
        out_shape=jax.ShapeDtypeStruct((_NG, _H, _D), jnp.float32),
    )(block_tables, context_lens, qg,
      *([key_cache] * _TB), *([value_cache] * _TB))


def kernel(query, key, value, key_cache, value_cache, slot_mapping,
           block_tables, context_lens):
    n_tok = query.shape[0]
    start = _NP * _PL

    qp = query[:start].reshape(_NP, _PL, _HD)
    kp = key[:start].reshape(_NP, _PL, _HD)
    vp = value[:start].reshape(_NP, _PL, _HD)
    out_p = _prompt_attention(qp, kp, vp).reshape(start, _HD)

    nslots = _NB * _BS
    inv = jnp.zeros((nslots,), jnp.int32).at[slot_mapping].set(
        jnp.arange(n_tok, dtype=jnp.int32))
    valid = jnp.zeros((nslots,), jnp.int32).at[slot_mapping].set(1)

    kc_upd, vc_upd = _cache_update(key_cache, value_cache, key, value,
                                   inv, valid)

    qg = query[start:].reshape(_NG, _H, _D)
    out_g = _gen_attention(qg, kc_upd, vc_upd,
                           block_tables.astype(jnp.int32), context_lens)

    return jnp.concatenate([out_p, out_g.reshape(_NG, _HD)], axis=0)
